# Initial kernel scaffold; baseline (speedup 1.0000x reference)
import jax
import jax.numpy as jnp
from jax.experimental import pallas as pl


def kernel(x, edge_index, W, att_src, att_dst, bias):
    raise NotImplementedError("write your pallas kernel here")


# speedup vs baseline: 19.7508x; 19.7508x over previous
import functools
import jax
import jax.numpy as jnp
from jax import lax
from jax.experimental import pallas as pl
from jax.experimental.pallas import tpu as pltpu
from jax.experimental.pallas import tpu_sc as plsc

N = 10000
E = 320000
F = 128
NC = 2
NS = 16
NT = NC * NS
EPT = E // NT
K = 80
NCH = EPT // K
TILE_STRIDE = 624
TILE_COPY = 640
NEG_SLOPE = 0.2


def _vgather16(vec, idx):
    dnums = lax.GatherDimensionNumbers(
        offset_dims=(), collapsed_slice_dims=(0,), start_index_map=(0,))
    return lax.gather(vec, idx[:, None], dnums, (1,),
                      mode=lax.GatherScatterMode.PROMISE_IN_BOUNDS)


def _proj_body(x_ref, w_ref, asrc_ref, adst_ref, xw_ref, la_src_ref, la_dst_ref):
    xw = jnp.dot(x_ref[...], w_ref[...], preferred_element_type=jnp.float32)
    xw_ref[...] = xw
    la_src_ref[...] = jnp.sum(xw * asrc_ref[...], axis=1, keepdims=True)
    la_dst_ref[...] = jnp.sum(xw * adst_ref[...], axis=1, keepdims=True)


def _project(x, W, att_src, att_dst):
    blk = 1000
    grid = N // blk
    return pl.pallas_call(
        _proj_body,
        grid=(grid,),
        in_specs=[
            pl.BlockSpec((blk, F), lambda i: (i, 0)),
            pl.BlockSpec((F, F), lambda i: (0, 0)),
            pl.BlockSpec((1, F), lambda i: (0, 0)),
            pl.BlockSpec((1, F), lambda i: (0, 0)),
        ],
        out_specs=[
            pl.BlockSpec((blk, F), lambda i: (i, 0)),
            pl.BlockSpec((blk, 1), lambda i: (i, 0)),
            pl.BlockSpec((blk, 1), lambda i: (i, 0)),
        ],
        out_shape=[
            jax.ShapeDtypeStruct((N, F), jnp.float32),
            jax.ShapeDtypeStruct((N, 1), jnp.float32),
            jax.ShapeDtypeStruct((N, 1), jnp.float32),
        ],
    )(x, W, att_src, att_dst)


def _coef_body(src_hbm, dst_hbm, asrc_hbm, adst_hbm, ex_hbm, den_hbm,
               srcv, dstv, asv, adv, exbuf, denv):
    cid = lax.axis_index("c")
    sid = lax.axis_index("s")
    wid = cid * NS + sid

    zeros16 = jnp.zeros((16,), jnp.float32)
    lane = lax.iota(jnp.int32, 16)

    def zden(r, _):
        denv[pl.ds(r * 16, 16)] = zeros16
        return _
    lax.fori_loop(0, N // 16, zden, None)

    pltpu.sync_copy(src_hbm.at[wid], srcv)
    pltpu.sync_copy(dst_hbm.at[wid], dstv)
    pltpu.sync_copy(asrc_hbm, asv)
    pltpu.sync_copy(adst_hbm, adv)

    def chunk(j, _):
        for k in range(K // 16):
            s16 = srcv[j, pl.ds(k * 16, 16)]
            d16 = dstv[j, pl.ds(k * 16, 16)]
            alpha = plsc.load_gather(asv, [s16]) + plsc.load_gather(adv, [d16])
            alpha = jnp.where(alpha >= 0, alpha, NEG_SLOPE * alpha)
            ex = jnp.exp(alpha)
            exbuf[j, pl.ds(k * 16, 16)] = ex

            keys, vals = plsc.sort_key_val(d16, ex)
            c = plsc.cumsum(vals)
            nxt = jnp.minimum(lane + 1, 15)
            keys_next = _vgather16(keys, nxt)
            run_end = keys != keys_next
            m_last = run_end | (lane == 15)
            plsc.addupdate_scatter(denv, [keys], c, mask=m_last)
            plsc.addupdate_scatter(denv, [keys_next], -c, mask=run_end)
        return _
    lax.fori_loop(0, NCH, chunk, None)

    pltpu.sync_copy(exbuf, ex_hbm.at[wid])
    pltpu.sync_copy(denv, den_hbm.at[wid])


def _coef_pass(src3, dst3, a_src, a_dst):
    mesh = plsc.VectorSubcoreMesh(core_axis_name="c", subcore_axis_name="s")
    fn = pl.kernel(
        _coef_body,
        out_type=[
            jax.ShapeDtypeStruct((NT, NCH, K), jnp.float32),
            jax.ShapeDtypeStruct((NT, N), jnp.float32),
        ],
        mesh=mesh,
        scratch_types=[
            pltpu.VMEM((NCH, K), jnp.int32),
            pltpu.VMEM((NCH, K), jnp.int32),
            pltpu.VMEM((N,), jnp.float32),
            pltpu.VMEM((N,), jnp.float32),
            pltpu.VMEM((NCH, K), jnp.float32),
            pltpu.VMEM((N,), jnp.float32),
        ],
        compiler_params=pltpu.CompilerParams(needs_layout_passes=False),
    )
    return fn(src3, dst3, a_src, a_dst)


def _agg_body(xw_hbm, src_hbm, dst_hbm, ex_hbm, out_hbm,
              sidx, didx, exb, gbuf, sbuf, acc, sem):
    cid = lax.axis_index("c")
    sid = lax.axis_index("s")
    wid = cid * NS + sid

    zeros16 = jnp.zeros((16,), jnp.float32)

    def zrow(r, _):
        for k in range(F // 16):
            sbuf[r, pl.ds(k * 16, 16)] = zeros16
        return _
    lax.fori_loop(0, K, zrow, None)
    for i in range(TILE_COPY // K):
        pltpu.sync_copy(sbuf, acc.at[pl.ds(sid * TILE_STRIDE + i * K, K)])
    plsc.subcore_barrier()

    def chunk(j, _):
        pltpu.sync_copy(src_hbm.at[wid, j], sidx)
        pltpu.sync_copy(dst_hbm.at[wid, j], didx)
        pltpu.sync_copy(ex_hbm.at[wid, j], exb)
        pltpu.async_copy(xw_hbm.at[sidx], gbuf, sem).wait()

        def scale_grp(g, _):
            ex16 = exb[pl.ds(g * 16, 16)]
            base = g * 16
            for rr in range(16):
                r = base + rr
                exs = ex16[rr]
                for k in range(F // 16):
                    sbuf[r, pl.ds(k * 16, 16)] = gbuf[r, pl.ds(k * 16, 16)] * exs
            return _
        lax.fori_loop(0, K // 16, scale_grp, None)

        pltpu.sync_copy(sbuf, acc.at[didx], add=True)
        return _
    lax.fori_loop(0, NCH, chunk, None)

    plsc.subcore_barrier()
    pltpu.sync_copy(acc.at[pl.ds(sid * TILE_STRIDE, TILE_COPY)],
                    out_hbm.at[cid, pl.ds(sid * TILE_STRIDE, TILE_COPY)])


def _agg_pass(xw, src3, dst3, ex3):
    mesh = plsc.VectorSubcoreMesh(core_axis_name="c", subcore_axis_name="s")
    fn = pl.kernel(
        _agg_body,
        out_type=jax.ShapeDtypeStruct((NC, N, F), jnp.float32),
        mesh=mesh,
        scratch_types=[
            pltpu.VMEM((K,), jnp.int32),
            pltpu.VMEM((K,), jnp.int32),
            pltpu.VMEM((K,), jnp.float32),
            pltpu.VMEM((K, F), jnp.float32),
            pltpu.VMEM((K, F), jnp.float32),
            pltpu.VMEM_SHARED((N, F), jnp.float32),
            pltpu.SemaphoreType.DMA,
        ],
        compiler_params=pltpu.CompilerParams(needs_layout_passes=False),
    )
    return fn(xw, src3, dst3, ex3)


def _combine_body(acc_ref, den_ref, bias_ref, out_ref):
    num = acc_ref[0] + acc_ref[1]
    den = jnp.sum(den_ref[...], axis=0)
    out_ref[...] = num / (den[:, None] + 1e-16) + bias_ref[...]


def _combine(accs, dens, bias):
    return pl.pallas_call(
        _combine_body,
        out_shape=jax.ShapeDtypeStruct((N, F), jnp.float32),
    )(accs, dens, bias)


def kernel(x, edge_index, W, att_src, att_dst, bias):
    att_src2 = att_src.reshape(1, F)
    att_dst2 = att_dst.reshape(1, F)
    xw, la_src, la_dst = _project(x, W, att_src2, att_dst2)
    src3 = edge_index[0].reshape(NT, NCH, K)
    dst3 = edge_index[1].reshape(NT, NCH, K)
    ex3, dens = _coef_pass(src3, dst3,
                           la_src.reshape(N), la_dst.reshape(N))
    accs = _agg_pass(xw, src3, dst3, ex3)
    return _combine(accs, dens, bias.reshape(1, F))
